# baseline (speedup 1.0000x reference)
import functools
import math

import jax
import jax.numpy as jnp
from jax import lax
from jax.experimental import pallas as pl
from jax.experimental.pallas import tpu as pltpu
from jax.experimental.pallas import tpu_sc as plsc


@functools.lru_cache(maxsize=None)
def _build_gather_scale(N: int, V: int, D: int):
    info = plsc.get_sparse_core_info()
    NC, NS, L = info.num_cores, info.num_subcores, info.num_lanes
    NW = NC * NS
    assert N % NW == 0 and D % L == 0
    b_per_w = N // NW
    C = 32
    NBUF = 3
    DEPTH = 1
    assert b_per_w % C == 0 and C % 8 == 0
    assert NBUF > DEPTH + 1
    n_chunks = b_per_w // C
    scale = math.sqrt(D)

    mesh = plsc.VectorSubcoreMesh(core_axis_name="c", subcore_axis_name="s")

    @functools.partial(
        pl.kernel,
        mesh=mesh,
        out_type=jax.ShapeDtypeStruct((N, D), jnp.float32),
        scratch_types=[
            pltpu.VMEM((n_chunks, C), jnp.int32),
            pltpu.VMEM((NBUF, C, D), jnp.float32),
        ]
        + [pltpu.SemaphoreType.DMA] * (2 * NBUF),
    )
    def k(idx_hbm, table_hbm, out_hbm, idx_v, rows_v, *sems):
        gsem = sems[:NBUF]
        wsem = sems[NBUF:]
        wid = lax.axis_index("s") * NC + lax.axis_index("c")
        base = wid * b_per_w
        pltpu.sync_copy(idx_hbm.at[wid], idx_v)

        def start_gather(g, b):
            return pltpu.async_copy(
                table_hbm.at[idx_v.at[g]],
                rows_v.at[b],
                gsem[b],
            )

        H = C // 2

        def start_write(g, h, b):
            return pltpu.async_copy(
                rows_v.at[b].at[pl.ds(h * H, H)],
                out_hbm.at[pl.ds(base + g * C + h * H, H)],
                wsem[b],
            )

        n_sl = D // L
        gh = {}
        wh = {}
        for g0 in range(DEPTH):
            gh[g0] = start_gather(g0, g0 % NBUF)
        for g in range(n_chunks):
            b = g % NBUF
            gn = g + DEPTH
            if gn < n_chunks:
                bn = gn % NBUF
                for h in range(2):
                    if (gn - NBUF, h) in wh:
                        wh.pop((gn - NBUF, h)).wait()
                gh[gn] = start_gather(gn, bn)
            gh.pop(g).wait()

            for h in range(2):

                @plsc.parallel_loop(0, H * n_sl, unroll=4)
                def _(i, b=b, h=h):
                    r = h * H + i // n_sl
                    sl = pl.ds((i % n_sl) * L, L)
                    rows_v[b, r, sl] = rows_v[b, r, sl] * scale

                wh[(g, h)] = start_write(g, h, b)
        for key in sorted(wh):
            wh.pop(key).wait()

    def run(idx_flat, table):
        return k(idx_flat.reshape(NW, n_chunks, C), table)

    return run


def kernel(x, embedding):
    B, S = x.shape
    V, D = embedding.shape
    N = B * S
    idx = x.reshape(N).astype(jnp.int32)
    out = _build_gather_scale(N, V, D)(idx, embedding)
    return out.reshape(B, S, D)


# speedup vs baseline: 1.0142x; 1.0142x over previous
import functools
import math

import jax
import jax.numpy as jnp
from jax import lax
from jax.experimental import pallas as pl
from jax.experimental.pallas import tpu as pltpu
from jax.experimental.pallas import tpu_sc as plsc


@functools.lru_cache(maxsize=None)
def _build_gather_scale(N: int, V: int, D: int):
    info = plsc.get_sparse_core_info()
    NC, NS, L = info.num_cores, info.num_subcores, info.num_lanes
    NW = NC * NS
    assert N % NW == 0 and D % L == 0
    b_per_w = N // NW
    C = 32
    NBUF = 3
    DEPTH = 1
    assert b_per_w % C == 0 and C % 8 == 0
    assert NBUF > DEPTH + 1
    n_chunks = b_per_w // C
    scale = math.sqrt(D)

    mesh = plsc.VectorSubcoreMesh(core_axis_name="c", subcore_axis_name="s")

    @functools.partial(
        pl.kernel,
        mesh=mesh,
        out_type=jax.ShapeDtypeStruct((N, D), jnp.float32),
        scratch_types=[
            pltpu.VMEM((n_chunks, C), jnp.int32),
            pltpu.VMEM((NBUF, C, D), jnp.float32),
        ]
        + [pltpu.SemaphoreType.DMA] * (2 * NBUF),
    )
    def k(idx_hbm, table_hbm, out_hbm, idx_v, rows_v, *sems):
        gsem = sems[:NBUF]
        wsem = sems[NBUF:]
        wid = lax.axis_index("s") * NC + lax.axis_index("c")
        base = wid * b_per_w
        pltpu.sync_copy(idx_hbm.at[wid], idx_v)

        def start_gather(g, b):
            return pltpu.async_copy(
                table_hbm.at[idx_v.at[g]],
                rows_v.at[b],
                gsem[b],
            )

        def start_write(g, b):
            return pltpu.async_copy(
                rows_v.at[b],
                out_hbm.at[pl.ds(base + g * C, C)],
                wsem[b],
            )

        n_sl = D // L
        gh = {}
        wh = {}
        for g0 in range(DEPTH):
            gh[g0] = start_gather(g0, g0 % NBUF)
        for g in range(n_chunks):
            b = g % NBUF
            gn = g + DEPTH
            if gn < n_chunks:
                bn = gn % NBUF
                if gn - NBUF in wh:
                    wh.pop(gn - NBUF).wait()
                gh[gn] = start_gather(gn, bn)
            gh.pop(g).wait()

            @plsc.parallel_loop(0, C * n_sl, unroll=4)
            def _(i, b=b):
                r = i // n_sl
                sl = pl.ds((i % n_sl) * L, L)
                rows_v[b, r, sl] = rows_v[b, r, sl] * scale

            wh[g] = start_write(g, b)
        for g in sorted(wh):
            wh.pop(g).wait()

    def run(idx_flat, table):
        return k(idx_flat.reshape(NW, n_chunks, C), table)

    return run


def kernel(x, embedding):
    B, S = x.shape
    V, D = embedding.shape
    N = B * S
    idx = x.reshape(N).astype(jnp.int32)
    out = _build_gather_scale(N, V, D)(idx, embedding)
    return out.reshape(B, S, D)


# speedup vs baseline: 1.0172x; 1.0030x over previous
import functools
import math

import jax
import jax.numpy as jnp
from jax import lax
from jax.experimental import pallas as pl
from jax.experimental.pallas import tpu as pltpu
from jax.experimental.pallas import tpu_sc as plsc


@functools.lru_cache(maxsize=None)
def _build_gather_scale(N: int, V: int, D: int):
    info = plsc.get_sparse_core_info()
    NC, NS, L = info.num_cores, info.num_subcores, info.num_lanes
    NW = NC * NS
    assert N % NW == 0 and D % L == 0
    b_per_w = N // NW
    C = 32
    NBUF = 3
    DEPTH = 1
    assert b_per_w % C == 0 and C % 8 == 0
    assert NBUF > DEPTH + 1
    n_chunks = b_per_w // C
    scale = math.sqrt(D)

    mesh = plsc.VectorSubcoreMesh(core_axis_name="c", subcore_axis_name="s")

    @functools.partial(
        pl.kernel,
        mesh=mesh,
        out_type=jax.ShapeDtypeStruct((N, D), jnp.float32),
        scratch_types=[
            pltpu.VMEM((n_chunks, C), jnp.int32),
            pltpu.VMEM((NBUF, C, D), jnp.float32),
        ]
        + [pltpu.SemaphoreType.DMA] * (2 * NBUF),
    )
    def k(idx_hbm, table_hbm, out_hbm, idx_v, rows_v, *sems):
        gsem = sems[:NBUF]
        wsem = sems[NBUF:]
        wid = lax.axis_index("s") * NC + lax.axis_index("c")
        base = wid * b_per_w
        pltpu.sync_copy(idx_hbm.at[wid], idx_v)

        def start_gather(g, b):
            return pltpu.async_copy(
                table_hbm.at[idx_v.at[g]],
                rows_v.at[b],
                gsem[b],
            )

        def start_write(g, b):
            return pltpu.async_copy(
                rows_v.at[b],
                out_hbm.at[pl.ds(base + g * C, C)],
                wsem[b],
            )

        n_sl = D // L
        gh = {}
        wh = {}
        for g0 in range(DEPTH):
            gh[g0] = start_gather(g0, g0 % NBUF)
        for g in range(n_chunks):
            b = g % NBUF
            gn = g + DEPTH
            if gn < n_chunks:
                bn = gn % NBUF
                if gn - NBUF in wh:
                    wh.pop(gn - NBUF).wait()
                gh[gn] = start_gather(gn, bn)
            gh.pop(g).wait()

            @plsc.parallel_loop(0, C * n_sl, unroll=8)
            def _(i, b=b):
                r = i // n_sl
                sl = pl.ds((i % n_sl) * L, L)
                rows_v[b, r, sl] = rows_v[b, r, sl] * scale

            wh[g] = start_write(g, b)
        for g in sorted(wh):
            wh.pop(g).wait()

    def run(idx_flat, table):
        return k(idx_flat.reshape(NW, n_chunks, C), table)

    return run


def kernel(x, embedding):
    B, S = x.shape
    V, D = embedding.shape
    N = B * S
    idx = x.reshape(N).astype(jnp.int32)
    out = _build_gather_scale(N, V, D)(idx, embedding)
    return out.reshape(B, S, D)
